# baseline (speedup 1.0000x reference)
import functools

import jax
import jax.numpy as jnp
from jax import lax
from jax.experimental import pallas as pl
from jax.experimental.pallas import tpu as pltpu
from jax.experimental.pallas import tpu_sc as plsc

NC, NS, LANES = 2, 16, 16
NW = NC * NS
D = 64
CHUNK = 128


def _emb_body(T, idx_hbm, tt_hbm, post_hbm, out_hbm,
              idxr_v, gidx_v, h64_v, g_a, g_b, p_v, o_a, o_b,
              gs_a, gs_b, os_a, os_b):
    wid = lax.axis_index("s") * NC + lax.axis_index("c")
    n_tchunks = T // CHUNK
    n_units = 2 * n_tchunks

    for k in range(2):
        for m in range(n_tchunks):
            pltpu.sync_copy(idx_hbm.at[2 * wid + k, pl.ds(m * CHUNK, CHUNK)],
                            idxr_v.at[k, m])

    def prep(i, _):
        r = i // (n_tchunks * 8)
        m = (i // 8) % n_tchunks
        s = (i % 8) * LANES
        v = idxr_v[r, m, pl.ds(s, LANES)]
        gidx_v[r, m, pl.ds(s, LANES)] = lax.shift_right_logical(v, 1)
        h64_v[r, m, pl.ds(s, LANES)] = lax.shift_left(
            lax.bitwise_and(v, 1), 6)
        return 0

    lax.fori_loop(0, 2 * n_tchunks * 8, prep, 0)

    g_bufs, g_sems = (g_a, g_b), (gs_a, gs_b)
    o_bufs, o_sems = (o_a, o_b), (os_a, os_b)

    def gather_pair(u):
        k, tc = u % 2, u // 2
        return (tt_hbm.at[gidx_v.at[k, tc]], g_bufs[u % 2], g_sems[u % 2])

    def store_pair(u):
        k, tc = u % 2, u // 2
        b = 2 * wid + k
        return (o_bufs[u % 2], out_hbm.at[b, :, pl.ds(tc * CHUNK, CHUNK)],
                o_sems[u % 2])

    pltpu.async_copy(*gather_pair(0))
    for u in range(n_units):
        k, tc = u % 2, u // 2
        g_buf, o_buf = g_bufs[u % 2], o_bufs[u % 2]
        pltpu.make_async_copy(*gather_pair(u)).wait()
        if u + 1 < n_units:
            pltpu.async_copy(*gather_pair(u + 1))
        if k == 0:
            pltpu.sync_copy(post_hbm.at[:, pl.ds(tc * CHUNK, CHUNK)], p_v)
        if u >= 2:
            pltpu.make_async_copy(*store_pair(u - 2)).wait()

        def assemble(d, _):
            for g in range(CHUNK // LANES):
                tb = g * LANES
                rows = tb + lax.broadcasted_iota(jnp.int32, (LANES,), 0)
                cols = h64_v[k, tc, pl.ds(tb, LANES)] + d
                val = plsc.load_gather(g_buf, [rows, cols])
                o_buf[d, pl.ds(tb, LANES)] = val + p_v[d, pl.ds(tb, LANES)]
            return 0

        lax.fori_loop(0, D, assemble, 0)
        pltpu.async_copy(*store_pair(u))
    pltpu.make_async_copy(*store_pair(n_units - 2)).wait()
    pltpu.make_async_copy(*store_pair(n_units - 1)).wait()


def kernel(idx, token_table, position_table):
    B, T = idx.shape
    V, d = token_table.shape
    assert d == D and (B * T) % (NW * CHUNK) == 0 and T % CHUNK == 0
    assert V % 2 == 0

    tt2 = token_table.reshape(V // 2, 2 * D)
    post = position_table.T
    idx32 = idx.astype(jnp.int32)

    mesh = plsc.VectorSubcoreMesh(core_axis_name="c", subcore_axis_name="s")
    body = functools.partial(_emb_body, T)
    out_bdt = pl.kernel(
        body,
        out_type=jax.ShapeDtypeStruct((B, D, T), jnp.float32),
        mesh=mesh,
        compiler_params=pltpu.CompilerParams(use_tc_tiling_on_sc=True,
                                             needs_layout_passes=False),
        scratch_types=[
            pltpu.VMEM((2, T // CHUNK, CHUNK), jnp.int32),
            pltpu.VMEM((2, T // CHUNK, CHUNK), jnp.int32),
            pltpu.VMEM((2, T // CHUNK, CHUNK), jnp.int32),
            pltpu.VMEM((CHUNK, 2 * D), jnp.float32),
            pltpu.VMEM((CHUNK, 2 * D), jnp.float32),
            pltpu.VMEM((D, CHUNK), jnp.float32),
            pltpu.VMEM((D, CHUNK), jnp.float32),
            pltpu.VMEM((D, CHUNK), jnp.float32),
            pltpu.SemaphoreType.DMA,
            pltpu.SemaphoreType.DMA,
            pltpu.SemaphoreType.DMA,
            pltpu.SemaphoreType.DMA,
        ],
    )(idx32, tt2, post)
    return out_bdt.transpose(0, 2, 1)
